# baseline (speedup 1.0000x reference)
import functools

import jax
import jax.numpy as jnp
from jax import lax
from jax.experimental import pallas as pl
from jax.experimental.pallas import tpu as pltpu
from jax.experimental.pallas import tpu_sc as plsc

_NC = 2
_NS = 16
_NW = _NC * _NS
_CHUNK = 128


def kernel(x, table):
    B, H = x.shape
    V, D = table.shape
    n = B * H
    per_w = n // _NW
    nchunk = per_w // _CHUNK
    idx = x.reshape(_NW, nchunk, _CHUNK).astype(jnp.int32)

    mesh = plsc.VectorSubcoreMesh(core_axis_name="c", subcore_axis_name="s")

    @functools.partial(
        pl.kernel,
        mesh=mesh,
        compiler_params=pltpu.CompilerParams(use_tc_tiling_on_sc=False),
        out_type=jax.ShapeDtypeStruct((n, D), jnp.float32),
        scratch_types=[
            pltpu.VMEM((nchunk, _CHUNK), jnp.int32),
            pltpu.VMEM((_CHUNK, D), jnp.float32),
            pltpu.SemaphoreType.DMA,
        ],
    )
    def _embed(idx_hbm, table_hbm, out_hbm, idx_v, rows_v, sem):
        wid = lax.axis_index("s") * _NC + lax.axis_index("c")
        base = wid * per_w
        pltpu.sync_copy(idx_hbm.at[wid], idx_v)

        def chunk_body(c, carry):
            pltpu.async_copy(table_hbm.at[idx_v.at[c]], rows_v, sem).wait()
            pltpu.sync_copy(rows_v, out_hbm.at[pl.ds(base + c * _CHUNK, _CHUNK)])
            return carry

        lax.fori_loop(0, nchunk, chunk_body, 0)

    out = _embed(idx, table)
    return out.reshape(B, H, D)


# speedup vs baseline: 1.3615x; 1.3615x over previous
import functools

import jax
import jax.numpy as jnp
from jax import lax
from jax.experimental import pallas as pl
from jax.experimental.pallas import tpu as pltpu
from jax.experimental.pallas import tpu_sc as plsc

_NC = 2
_NS = 16
_NW = _NC * _NS
_CHUNK = 128
_NBUF = 4
_NSTAGE = 2
_LANES = 16
_RUNROLL = 4


def kernel(x, table):
    B, H = x.shape
    V, D = table.shape
    n = B * H
    per_w = n // _NW
    nchunk = per_w // _CHUNK
    idx = x.reshape(_NW, nchunk, _CHUNK).astype(jnp.int32)
    table_wide = jnp.pad(table, ((0, 0), (0, 128 - D)))

    mesh = plsc.VectorSubcoreMesh(core_axis_name="c", subcore_axis_name="s")

    @functools.partial(
        pl.kernel,
        mesh=mesh,
        out_type=jax.ShapeDtypeStruct((n, D), jnp.float32),
        scratch_types=[
            pltpu.VMEM((nchunk, _CHUNK), jnp.int32),
            *[pltpu.VMEM((_CHUNK, 128), jnp.float32) for _ in range(_NBUF)],
            *[pltpu.VMEM((_CHUNK, D), jnp.float32) for _ in range(_NSTAGE)],
            *[pltpu.SemaphoreType.DMA for _ in range(_NBUF + _NSTAGE)],
        ],
    )
    def _embed(idx_hbm, table_hbm, out_hbm, idx_v, *bufs_and_sems):
        rows = bufs_and_sems[:_NBUF]
        stage = bufs_and_sems[_NBUF:_NBUF + _NSTAGE]
        gsem = bufs_and_sems[_NBUF + _NSTAGE:2 * _NBUF + _NSTAGE]
        ssem = bufs_and_sems[2 * _NBUF + _NSTAGE:]
        wid = lax.axis_index("s") * _NC + lax.axis_index("c")
        base = wid * per_w
        pltpu.sync_copy(idx_hbm.at[wid], idx_v)

        def gather_copy(c, b):
            return pltpu.make_async_copy(
                table_hbm.at[idx_v.at[c]], rows[b], gsem[b]
            )

        def store_copy(c, s):
            return pltpu.make_async_copy(
                stage[s],
                out_hbm.at[pl.ds(base + c * _CHUNK, _CHUNK)],
                ssem[s],
            )

        def compact(b, s):
            def row_body(r0, carry):
                for dr in range(_RUNROLL):
                    r = r0 * _RUNROLL + dr
                    for k in range(D // _LANES):
                        stage[s][r, pl.ds(k * _LANES, _LANES)] = (
                            rows[b][r, pl.ds(k * _LANES, _LANES)]
                        )
                return carry
            lax.fori_loop(0, _CHUNK // _RUNROLL, row_body, 0)

        def process(c, b, do_swait, do_prefetch):
            s = b % _NSTAGE
            gather_copy(c, b).wait()
            if do_swait:
                store_copy(c - _NSTAGE, s).wait()
            compact(b, s)
            store_copy(c, s).start()
            if do_prefetch:
                gather_copy(c + _NBUF, b).start()

        for b in range(_NBUF):
            gather_copy(b, b).start()
        for b in range(_NBUF):
            process(b, b, b >= _NSTAGE, True)

        def body(g, carry):
            for b in range(_NBUF):
                process(g * _NBUF + b, b, True, True)
            return carry

        lax.fori_loop(1, nchunk // _NBUF - 1, body, 0)

        for b in range(_NBUF):
            process(nchunk - _NBUF + b, b, True, False)
        for s in range(_NSTAGE):
            store_copy(nchunk - _NSTAGE + s, s).wait()

    out = _embed(idx, table_wide)
    return out.reshape(B, H, D)
